# Initial kernel scaffold; baseline (speedup 1.0000x reference)
import functools

import jax
import jax.numpy as jnp
from jax import lax
from jax.experimental import pallas as pl
from jax.experimental.pallas import tpu as pltpu
from jax.experimental.pallas import tpu_sc as plsc

N_NODES = 50000
N_EDGES = 1600000
D_EDGE = 16
NUM_GRAPHS = 128

_info = plsc.get_sparse_core_info()
_NC = _info.num_cores
_NS = _info.num_subcores
_L = _info.num_lanes
_NW = _NC * _NS
_EPW = N_EDGES // _NW
_C = 1000
_NCHUNK = _EPW // _C


def _sc_body(src_hbm, attr_hbm, batch_hbm, sums_hbm, cnts_hbm,
             batch_v, src_v, attr_v, acc_v, cnt_v):
    wid = lax.axis_index("s") * _NC + lax.axis_index("c")
    base = wid * _EPW

    pltpu.sync_copy(batch_hbm, batch_v)

    zeros = jnp.zeros((_L,), jnp.float32)

    def zrow(g, carry):
        acc_v[g, :] = zeros
        return carry
    lax.fori_loop(0, NUM_GRAPHS, zrow, 0)

    def zcnt(k, carry):
        cnt_v[pl.ds(k * _L, _L)] = zeros
        return carry
    lax.fori_loop(0, NUM_GRAPHS // _L, zcnt, 0)

    iota = lax.iota(jnp.int32, _L)
    ones = jnp.ones((_L,), jnp.float32)

    def chunk_body(ci, carry):
        cbase = base + ci * _C
        pltpu.sync_copy(src_hbm.at[pl.ds(cbase, _C)], src_v)
        pltpu.sync_copy(attr_hbm.at[pl.ds(cbase * D_EDGE, _C * D_EDGE)], attr_v)

        def grp(k, c2):
            sv = src_v[pl.ds(k * _L, _L)]
            gb = plsc.load_gather(batch_v, [sv])
            plsc.addupdate_scatter(cnt_v, [gb], ones)
            for j in range(_L):
                gs = jnp.take(gb, jnp.full((_L,), j, jnp.int32),
                              mode=lax.GatherScatterMode.PROMISE_IN_BOUNDS)
                row = attr_v[pl.ds((k * _L + j) * D_EDGE, D_EDGE)]
                plsc.addupdate_scatter(acc_v, [gs, iota], row)
            return c2
        lax.fori_loop(0, _C // _L, grp, 0)
        return carry
    lax.fori_loop(0, _NCHUNK, chunk_body, 0)

    pltpu.sync_copy(acc_v, sums_hbm.at[wid])
    pltpu.sync_copy(cnt_v, cnts_hbm.at[wid])


_sc_pool = functools.partial(
    pl.kernel,
    mesh=plsc.VectorSubcoreMesh(core_axis_name="c", subcore_axis_name="s"),
    out_type=[
        jax.ShapeDtypeStruct((_NW, NUM_GRAPHS, D_EDGE), jnp.float32),
        jax.ShapeDtypeStruct((_NW, NUM_GRAPHS), jnp.float32),
    ],
    scratch_types=[
        pltpu.VMEM((N_NODES,), jnp.int32),
        pltpu.VMEM((_C,), jnp.int32),
        pltpu.VMEM((_C * D_EDGE,), jnp.float32),
        pltpu.VMEM((NUM_GRAPHS, D_EDGE), jnp.float32),
        pltpu.VMEM((NUM_GRAPHS,), jnp.float32),
    ],
)(_sc_body)


def _fin_body(sums_ref, cnts_ref, out_ref):
    s = jnp.sum(sums_ref[...], axis=0)
    c = jnp.sum(cnts_ref[...], axis=0)[:, None]
    out_ref[...] = jnp.where(c > 0, s / jnp.maximum(c, 1.0), 0.0)


_finalize = pl.pallas_call(
    _fin_body,
    out_shape=jax.ShapeDtypeStruct((NUM_GRAPHS, D_EDGE), jnp.float32),
)


@jax.jit
def kernel(edge_index, edge_attr, batch):
    src = edge_index[0].astype(jnp.int32)
    attr_flat = edge_attr.reshape(-1)
    sums, cnts = _sc_pool(src, attr_flat, batch.astype(jnp.int32))
    return _finalize(sums, cnts)


# speedup vs baseline: 19.7940x; 19.7940x over previous
import functools

import jax
import jax.numpy as jnp
from jax import lax
from jax.experimental import pallas as pl
from jax.experimental.pallas import tpu as pltpu
from jax.experimental.pallas import tpu_sc as plsc

N_NODES = 50000
N_EDGES = 1600000
D_EDGE = 16
NUM_GRAPHS = 128

_info = plsc.get_sparse_core_info()
_NC = _info.num_cores
_NS = _info.num_subcores
_L = _info.num_lanes
_NW = _NC * _NS
_EPW = N_EDGES // _NW
_C = 2000
_NCHUNK = _EPW // _C


def _sc_body(src_hbm, attr_hbm, batch_hbm, sums_hbm, cnts_hbm,
             batch_v, src_v, attr_v, acc_v, cnt_v):
    wid = lax.axis_index("s") * _NC + lax.axis_index("c")
    base = wid * _EPW

    pltpu.sync_copy(batch_hbm, batch_v)

    zeros = jnp.zeros((_L,), jnp.float32)

    def zrow(g, carry):
        acc_v[g, :] = zeros
        return carry
    lax.fori_loop(0, NUM_GRAPHS, zrow, 0)

    def zcnt(k, carry):
        cnt_v[pl.ds(k * _L, _L)] = zeros
        return carry
    lax.fori_loop(0, NUM_GRAPHS // _L, zcnt, 0)

    iota = lax.iota(jnp.int32, _L)
    ones = jnp.ones((_L,), jnp.float32)
    lane0 = iota == 0
    _dnums = lax.GatherDimensionNumbers(
        offset_dims=(), collapsed_slice_dims=(0,), start_index_map=(0,))

    def _bcast_lane(vec, j):
        idx = jnp.full((_L, 1), j, jnp.int32)
        return lax.gather(vec, idx, _dnums, slice_sizes=(1,),
                          mode=lax.GatherScatterMode.PROMISE_IN_BOUNDS)

    def chunk_body(ci, carry):
        cbase = base + ci * _C
        pltpu.sync_copy(src_hbm.at[pl.ds(cbase, _C)], src_v)
        pltpu.sync_copy(attr_hbm.at[pl.ds(cbase * D_EDGE, _C * D_EDGE)], attr_v)

        def grp(k, c2):
            sv = src_v[pl.ds(k * _L, _L)]
            gb = plsc.load_gather(batch_v, [sv])
            for j in range(_L):
                gs = _bcast_lane(gb, j)
                row = attr_v[pl.ds((k * _L + j) * D_EDGE, D_EDGE)]
                plsc.addupdate_scatter(acc_v, [gs, iota], row)
                plsc.addupdate_scatter(cnt_v, [gs], ones, mask=lane0)
            return c2
        lax.fori_loop(0, _C // _L, grp, 0)
        return carry
    lax.fori_loop(0, _NCHUNK, chunk_body, 0)

    pltpu.sync_copy(acc_v, sums_hbm.at[wid])
    pltpu.sync_copy(cnt_v, cnts_hbm.at[wid])


_sc_pool = functools.partial(
    pl.kernel,
    mesh=plsc.VectorSubcoreMesh(core_axis_name="c", subcore_axis_name="s"),
    compiler_params=pltpu.CompilerParams(needs_layout_passes=False),
    out_type=[
        jax.ShapeDtypeStruct((_NW, NUM_GRAPHS, D_EDGE), jnp.float32),
        jax.ShapeDtypeStruct((_NW, NUM_GRAPHS), jnp.float32),
    ],
    scratch_types=[
        pltpu.VMEM((N_NODES,), jnp.int32),
        pltpu.VMEM((_C,), jnp.int32),
        pltpu.VMEM((_C * D_EDGE,), jnp.float32),
        pltpu.VMEM((NUM_GRAPHS, D_EDGE), jnp.float32),
        pltpu.VMEM((NUM_GRAPHS,), jnp.float32),
    ],
)(_sc_body)


def _fin_body(sums_ref, cnts_ref, out_ref):
    s = jnp.sum(sums_ref[...], axis=0)
    c = jnp.sum(cnts_ref[...], axis=0)[:, None]
    out_ref[...] = jnp.where(c > 0, s / jnp.maximum(c, 1.0), 0.0)


_finalize = pl.pallas_call(
    _fin_body,
    out_shape=jax.ShapeDtypeStruct((NUM_GRAPHS, D_EDGE), jnp.float32),
)


@jax.jit
def kernel(edge_index, edge_attr, batch):
    src = edge_index[0].astype(jnp.int32)
    attr_flat = edge_attr.reshape(-1)
    sums, cnts = _sc_pool(src, attr_flat, batch.astype(jnp.int32))
    return _finalize(sums, cnts)


# speedup vs baseline: 24.0602x; 1.2155x over previous
import functools

import jax
import jax.numpy as jnp
from jax import lax
from jax.experimental import pallas as pl
from jax.experimental.pallas import tpu as pltpu
from jax.experimental.pallas import tpu_sc as plsc

N_NODES = 50000
N_EDGES = 1600000
D_EDGE = 16
NUM_GRAPHS = 128

_info = plsc.get_sparse_core_info()
_NC = _info.num_cores
_NS = _info.num_subcores
_L = _info.num_lanes
_NW = _NC * _NS
_EPW = N_EDGES // _NW
_C = 2000
_NCHUNK = _EPW // _C


def _sc_body(src_hbm, attr_hbm, batch_hbm, sums_hbm, cnts_hbm,
             batch_v, src_v, attr_v, acc_v, cnt_v):
    wid = lax.axis_index("s") * _NC + lax.axis_index("c")
    base = wid * _EPW

    pltpu.sync_copy(batch_hbm, batch_v)

    zeros = jnp.zeros((_L,), jnp.float32)

    def zrow(g, carry):
        acc_v[pl.ds(g * _L, _L)] = zeros
        return carry
    lax.fori_loop(0, NUM_GRAPHS * D_EDGE // _L, zrow, 0)

    def zcnt(k, carry):
        cnt_v[pl.ds(k * _L, _L)] = zeros
        return carry
    lax.fori_loop(0, NUM_GRAPHS // _L, zcnt, 0)

    iota = lax.iota(jnp.int32, _L)
    ones = jnp.ones((_L,), jnp.float32)
    lane0 = iota == 0
    _dnums = lax.GatherDimensionNumbers(
        offset_dims=(), collapsed_slice_dims=(0,), start_index_map=(0,))

    def _bcast_lane(vec, j):
        idx = jnp.full((_L, 1), j, jnp.int32)
        return lax.gather(vec, idx, _dnums, slice_sizes=(1,),
                          mode=lax.GatherScatterMode.PROMISE_IN_BOUNDS)

    def chunk_body(ci, carry):
        cbase = base + ci * _C
        pltpu.sync_copy(src_hbm.at[pl.ds(cbase, _C)], src_v)
        pltpu.sync_copy(attr_hbm.at[pl.ds(cbase * D_EDGE, _C * D_EDGE)], attr_v)

        @plsc.parallel_loop(0, _C // _L, unroll=2)
        def grp(k):
            sv = src_v[pl.ds(k * _L, _L)]
            gb = plsc.load_gather(batch_v, [sv])
            plsc.addupdate_scatter(cnt_v, [gb], ones)
            gb16 = gb << 4
            for j in range(_L):
                idx = _bcast_lane(gb16, j) | iota
                row = attr_v[pl.ds((k * _L + j) * D_EDGE, D_EDGE)]
                plsc.addupdate_scatter(acc_v, [idx], row)
        return carry
    lax.fori_loop(0, _NCHUNK, chunk_body, 0)

    pltpu.sync_copy(acc_v, sums_hbm.at[wid])
    pltpu.sync_copy(cnt_v, cnts_hbm.at[wid])


_sc_pool = functools.partial(
    pl.kernel,
    mesh=plsc.VectorSubcoreMesh(core_axis_name="c", subcore_axis_name="s"),
    compiler_params=pltpu.CompilerParams(needs_layout_passes=False),
    out_type=[
        jax.ShapeDtypeStruct((_NW, NUM_GRAPHS * D_EDGE), jnp.float32),
        jax.ShapeDtypeStruct((_NW, NUM_GRAPHS), jnp.float32),
    ],
    scratch_types=[
        pltpu.VMEM((N_NODES,), jnp.int32),
        pltpu.VMEM((_C,), jnp.int32),
        pltpu.VMEM((_C * D_EDGE,), jnp.float32),
        pltpu.VMEM((NUM_GRAPHS * D_EDGE,), jnp.float32),
        pltpu.VMEM((NUM_GRAPHS,), jnp.float32),
    ],
)(_sc_body)


def _fin_body(sums_ref, cnts_ref, out_ref):
    s = jnp.sum(sums_ref[...], axis=0)
    c = jnp.sum(cnts_ref[...], axis=0)[:, None]
    out_ref[...] = jnp.where(c > 0, s / jnp.maximum(c, 1.0), 0.0)


_finalize = pl.pallas_call(
    _fin_body,
    out_shape=jax.ShapeDtypeStruct((NUM_GRAPHS, D_EDGE), jnp.float32),
)


@jax.jit
def kernel(edge_index, edge_attr, batch):
    src = edge_index[0].astype(jnp.int32)
    attr_flat = edge_attr.reshape(-1)
    sums, cnts = _sc_pool(src, attr_flat, batch.astype(jnp.int32))
    return _finalize(sums.reshape(_NW, NUM_GRAPHS, D_EDGE), cnts)


# speedup vs baseline: 101.3642x; 4.2129x over previous
import functools

import jax
import jax.numpy as jnp
from jax import lax
from jax.experimental import pallas as pl
from jax.experimental.pallas import tpu as pltpu
from jax.experimental.pallas import tpu_sc as plsc

N_NODES = 50000
N_EDGES = 1600000
D_EDGE = 16
NUM_GRAPHS = 128

_info = plsc.get_sparse_core_info()
_NC = _info.num_cores
_NS = _info.num_subcores
_L = _info.num_lanes
_NW = _NC * _NS
_C = 512
_EROWS = 2 * _C // 128
_NCHUNK = N_EDGES // _C
_TLO = _NCHUNK // _NW
_TREM = _NCHUNK % _NW


def _sc_body(ei_hbm, attr_hbm, batch_hbm, sums_hbm, cnts_hbm,
             batch_v, ev_v, attr_v, acc_v, cnt_v, sem_ev, sem_at):
    wid = lax.axis_index("s") * _NC + lax.axis_index("c")
    nt = _TLO + (wid < _TREM).astype(jnp.int32)

    pltpu.sync_copy(batch_hbm, batch_v)

    zeros = jnp.zeros((_L,), jnp.float32)

    def zrow(d, carry):
        for q in range(NUM_GRAPHS // _L):
            acc_v[d, pl.ds(q * _L, _L)] = zeros
        return carry
    lax.fori_loop(0, D_EDGE, zrow, 0)
    for q in range(NUM_GRAPHS // _L):
        cnt_v[pl.ds(q * _L, _L)] = zeros

    ones = jnp.ones((_L,), jnp.float32)
    dsplat = [jnp.full((_L,), d, jnp.int32) for d in range(D_EDGE)]

    def issue(t, b):
        ci = t * _NW + wid
        pltpu.async_copy(ei_hbm.at[pl.ds(ci * _EROWS, _EROWS), :],
                         ev_v.at[b], sem_ev.at[b])
        pltpu.async_copy(attr_hbm.at[:, pl.ds(ci * _C, _C)],
                         attr_v.at[b], sem_at.at[b])

    def wait(t, b):
        ci = t * _NW + wid
        pltpu.make_async_copy(ei_hbm.at[pl.ds(ci * _EROWS, _EROWS), :],
                              ev_v.at[b], sem_ev.at[b]).wait()
        pltpu.make_async_copy(attr_hbm.at[:, pl.ds(ci * _C, _C)],
                              attr_v.at[b], sem_at.at[b]).wait()

    def process(b):
        @plsc.parallel_loop(0, _C // _L, unroll=2)
        def grp(k):
            sv = ev_v[b, 2 * (k // 8), pl.ds((k % 8) * _L, _L)]
            gb = plsc.load_gather(batch_v, [sv])
            plsc.addupdate_scatter(cnt_v, [gb], ones)
            for d in range(D_EDGE):
                col = attr_v[b, d, pl.ds(k * _L, _L)]
                plsc.addupdate_scatter(acc_v, [dsplat[d], gb], col)

    issue(0, 0)
    issue(1, 1)

    def pair(tp, carry):
        for b in range(2):
            t = 2 * tp + b

            @pl.when(t < nt)
            def _():
                wait(t, b)
                process(b)

                @pl.when(t + 2 < nt)
                def _():
                    issue(t + 2, b)
        return carry
    lax.fori_loop(0, (_TLO + 2) // 2, pair, 0)

    pltpu.sync_copy(acc_v, sums_hbm.at[wid])
    pltpu.sync_copy(cnt_v, cnts_hbm.at[wid])


_sc_pool = functools.partial(
    pl.kernel,
    mesh=plsc.VectorSubcoreMesh(core_axis_name="c", subcore_axis_name="s"),
    compiler_params=pltpu.CompilerParams(needs_layout_passes=False),
    out_type=[
        jax.ShapeDtypeStruct((_NW, D_EDGE, NUM_GRAPHS), jnp.float32),
        jax.ShapeDtypeStruct((_NW, NUM_GRAPHS), jnp.float32),
    ],
    scratch_types=[
        pltpu.VMEM((N_NODES,), jnp.int32),
        pltpu.VMEM((2, _EROWS, 128), jnp.int32),
        pltpu.VMEM((2, D_EDGE, _C), jnp.float32),
        pltpu.VMEM((D_EDGE, NUM_GRAPHS), jnp.float32),
        pltpu.VMEM((NUM_GRAPHS,), jnp.float32),
        pltpu.SemaphoreType.DMA((2,)),
        pltpu.SemaphoreType.DMA((2,)),
    ],
)(_sc_body)


def _fin_body(sums_ref, cnts_ref, out_ref):
    s = jnp.sum(sums_ref[...], axis=0)
    c = jnp.sum(cnts_ref[...], axis=0)[None, :]
    m = jnp.where(c > 0, s / jnp.maximum(c, 1.0), 0.0)
    out_ref[...] = m.T


_finalize = pl.pallas_call(
    _fin_body,
    out_shape=jax.ShapeDtypeStruct((NUM_GRAPHS, D_EDGE), jnp.float32),
)


@jax.jit
def kernel(edge_index, edge_attr, batch):
    ei_view = (edge_index.astype(jnp.int32)
               .reshape(2, N_EDGES // 128, 128)
               .transpose(1, 0, 2)
               .reshape(2 * N_EDGES // 128, 128))
    attr_t = edge_attr.T
    sums, cnts = _sc_pool(ei_view, attr_t, batch.astype(jnp.int32))
    return _finalize(sums, cnts)


# speedup vs baseline: 130.3650x; 1.2861x over previous
import functools

import jax
import jax.numpy as jnp
from jax import lax
from jax.experimental import pallas as pl
from jax.experimental.pallas import tpu as pltpu
from jax.experimental.pallas import tpu_sc as plsc

N_NODES = 50000
N_EDGES = 1600000
D_EDGE = 16
NUM_GRAPHS = 128

_info = plsc.get_sparse_core_info()
_NC = _info.num_cores
_NS = _info.num_subcores
_L = _info.num_lanes
_NW = _NC * _NS
_C = 512
_EROWS = 2 * _C // 128
_NCHUNK = N_EDGES // _C
_TLO = _NCHUNK // _NW
_TREM = _NCHUNK % _NW
_NBUF = 2


def _sc_body(ei_hbm, attr_hbm, batch_hbm, sums_hbm, cnts_hbm,
             batch_v, ev_v, attr_v, acc_v, cnt_v, sem_ev, sem_at):
    wid = lax.axis_index("s") * _NC + lax.axis_index("c")
    nt = _TLO + (wid < _TREM).astype(jnp.int32)

    pltpu.sync_copy(batch_hbm, batch_v)

    zeros = jnp.zeros((_L,), jnp.float32)

    def zrow(d, carry):
        for q in range(NUM_GRAPHS // _L):
            acc_v[d, pl.ds(q * _L, _L)] = zeros
        return carry
    lax.fori_loop(0, D_EDGE, zrow, 0)
    for q in range(NUM_GRAPHS // _L):
        cnt_v[pl.ds(q * _L, _L)] = zeros

    ones = jnp.ones((_L,), jnp.float32)
    dsplat = [jnp.full((_L,), d, jnp.int32) for d in range(D_EDGE)]

    def issue(t, b):
        ci = t * _NW + wid
        pltpu.async_copy(ei_hbm.at[:, pl.ds(ci * _C, _C)],
                         ev_v.at[b], sem_ev.at[b])
        pltpu.async_copy(attr_hbm.at[:, pl.ds(ci * _C, _C)],
                         attr_v.at[b], sem_at.at[b])

    def wait(t, b):
        ci = t * _NW + wid
        pltpu.make_async_copy(ei_hbm.at[:, pl.ds(ci * _C, _C)],
                              ev_v.at[b], sem_ev.at[b]).wait()
        pltpu.make_async_copy(attr_hbm.at[:, pl.ds(ci * _C, _C)],
                              attr_v.at[b], sem_at.at[b]).wait()

    def process(b):
        @plsc.parallel_loop(0, _C // _L, unroll=2)
        def grp(k):
            sv = ev_v[b, 0, pl.ds(k * _L, _L)]
            gb = plsc.load_gather(batch_v, [sv])
            plsc.addupdate_scatter(cnt_v, [gb], ones)
            for d in range(D_EDGE):
                col = attr_v[b, d, pl.ds(k * _L, _L)]
                plsc.addupdate_scatter(acc_v, [dsplat[d], gb], col)

    for b in range(_NBUF):
        issue(b, b)

    def ring(tp, carry):
        for b in range(_NBUF):
            t = _NBUF * tp + b

            @pl.when(t < nt)
            def _():
                wait(t, b)
                process(b)

                @pl.when(t + _NBUF < nt)
                def _():
                    issue(t + _NBUF, b)
        return carry
    lax.fori_loop(0, (_TLO + _NBUF) // _NBUF, ring, 0)

    pltpu.sync_copy(acc_v, sums_hbm.at[wid])
    pltpu.sync_copy(cnt_v, cnts_hbm.at[wid])


_sc_pool = functools.partial(
    pl.kernel,
    mesh=plsc.VectorSubcoreMesh(core_axis_name="c", subcore_axis_name="s"),
    compiler_params=pltpu.CompilerParams(needs_layout_passes=False),
    out_type=[
        jax.ShapeDtypeStruct((_NW, D_EDGE, NUM_GRAPHS), jnp.float32),
        jax.ShapeDtypeStruct((_NW, NUM_GRAPHS), jnp.float32),
    ],
    scratch_types=[
        pltpu.VMEM((N_NODES,), jnp.int32),
        pltpu.VMEM((_NBUF, 2, _C), jnp.int32),
        pltpu.VMEM((_NBUF, D_EDGE, _C), jnp.float32),
        pltpu.VMEM((D_EDGE, NUM_GRAPHS), jnp.float32),
        pltpu.VMEM((NUM_GRAPHS,), jnp.float32),
        pltpu.SemaphoreType.DMA((_NBUF,)),
        pltpu.SemaphoreType.DMA((_NBUF,)),
    ],
)(_sc_body)


def _fin_body(sums_ref, cnts_ref, out_ref):
    s = jnp.sum(sums_ref[...], axis=0)
    c = jnp.sum(cnts_ref[...], axis=0)[None, :]
    m = jnp.where(c > 0, s / jnp.maximum(c, 1.0), 0.0)
    out_ref[...] = m.T


_finalize = pl.pallas_call(
    _fin_body,
    out_shape=jax.ShapeDtypeStruct((NUM_GRAPHS, D_EDGE), jnp.float32),
)


@jax.jit
def kernel(edge_index, edge_attr, batch):
    attr_t = edge_attr.T
    sums, cnts = _sc_pool(edge_index.astype(jnp.int32), attr_t,
                          batch.astype(jnp.int32))
    return _finalize(sums, cnts)
